# baseline (speedup 1.0000x reference)
import functools

import jax
import jax.numpy as jnp
import numpy as np
from jax import lax
from jax.experimental import pallas as pl
from jax.experimental.pallas import tpu as pltpu
from jax.experimental.pallas import tpu_sc as plsc

N = 10000
E = 160000
NUM_ELEMENTS = 4
C = 32
L = 9
NUM_BESSEL = 8
R_MAX = 5.0
NUM_GRAPHS = 16
N_ENERGIES = 3
AVG_NEIGH = 16.0
READ_DIM = 12
CL = C * L
HALF = CL // 2

BE = 2000
BN = 2000
GE = E // BE
GN = N // BN

_S = np.zeros((16, CL), np.float32)
_T = np.zeros((C, CL), np.float32)
for _l in range(L):
    for _c in range(C):
        _S[_l, _l * C + _c] = 1.0
        _T[_c, _l * C + _c] = 1.0
_W3PERM = np.array([c * L + l for l in range(L) for c in range(C)], np.int32)

_SQ3 = 1.7320508075688772
_SQ5 = 2.23606797749979
_SQ15 = 3.872983346207417


def _silu(x):
    return x * (1.0 / (1.0 + jnp.exp(-x)))


def _prep_body(pos_ref, attrs_ref, iv_ref, tbl_ref):
    pos = pos_ref[...]
    sp = attrs_ref[...] @ iv_ref[...]
    tbl_ref[...] = jnp.concatenate(
        [pos, sp, jnp.zeros((BN, 12), jnp.float32)], axis=1)


def _prep(positions, node_attrs, iota_col):
    return pl.pallas_call(
        _prep_body,
        grid=(GN,),
        in_specs=[
            pl.BlockSpec((BN, 3), lambda n: (n, 0)),
            pl.BlockSpec((BN, NUM_ELEMENTS), lambda n: (n, 0)),
            pl.BlockSpec((NUM_ELEMENTS, 1), lambda n: (0, 0)),
        ],
        out_specs=pl.BlockSpec((BN, 16), lambda n: (n, 0)),
        out_shape=jax.ShapeDtypeStruct((N, 16), jnp.float32),
    )(positions, node_attrs, iota_col)


def _geom_t(vx, vy, vz):
    r = jnp.sqrt(vx * vx + vy * vy + vz * vz + 1e-18)
    inv = 1.0 / r
    x, y, z = vx * inv, vy * inv, vz * inv
    ones = jnp.ones_like(x)
    YT = jnp.concatenate([
        ones, _SQ3 * x, _SQ3 * y, _SQ3 * z, _SQ15 * x * y, _SQ15 * y * z,
        0.5 * _SQ5 * (3.0 * z * z - 1.0), _SQ15 * x * z,
        0.5 * _SQ15 * (x * x - y * y),
        jnp.zeros((7, x.shape[1]), jnp.float32),
    ], axis=0)
    rr = jnp.maximum(r, 1e-9)
    xx = (jnp.pi / R_MAX) * rr
    s1 = jnp.sin(xx)
    c2 = 2.0 * jnp.cos(xx)
    sins = [s1, c2 * s1]
    for _ in range(NUM_BESSEL - 2):
        sins.append(c2 * sins[-1] - sins[-2])
    bes = jnp.sqrt(2.0 / R_MAX) * jnp.concatenate(sins, axis=0) / rr
    u = jnp.minimum(r * (1.0 / R_MAX), 1.0)
    u2 = u * u
    u5 = u2 * u2 * u
    f = 1.0 - 21.0 * u5 + 35.0 * u5 * u - 15.0 * u5 * u2
    fc = jnp.where(r < R_MAX, f, 0.0)
    return YT, bes * fc


def _radial(ef, w1, w2, w3):
    r1 = _silu(ef @ w1)
    r2 = _silu(r1 @ w2)
    return r2 @ w3


def _edge0_body(gs_ref, gr_ref, sh_ref, w1t_ref, w2t_ref, w3t_ref, st_ref,
                tt_ref, wupt_ref, m_ref, y_ref, ef_ref):
    gst = jnp.transpose(gs_ref[...][:, 0:4])
    grt = jnp.transpose(gr_ref[...][:, 0:3])
    sht = jnp.transpose(sh_ref[...])
    dps = grt - gst[0:3] + sht
    YT, efT = _geom_t(dps[0:1], dps[1:2], dps[2:3])
    r1 = _silu(w1t_ref[...] @ efT)
    r2 = _silu(w2t_ref[...] @ r1)
    RT = w3t_ref[...] @ r2
    spi = gst[3:4].astype(jnp.int32)
    oh = jnp.where(
        lax.broadcasted_iota(jnp.int32, (NUM_ELEMENTS, spi.shape[1]), 0)
        == spi, 1.0, 0.0)
    hT = wupt_ref[...] @ oh
    mT = RT * (tt_ref[...] @ hT) * (st_ref[...] @ YT)
    m = jnp.transpose(mT)
    m_ref[0] = m[:, :HALF]
    m_ref[1] = m[:, HALF:]
    y_ref[...] = jnp.transpose(YT)
    ef_ref[...] = jnp.transpose(efT)


def _edge0(gs, gr, shifts, w1t, w2t, w3t, s_t, t_t, wupt):
    return pl.pallas_call(
        _edge0_body,
        grid=(GE,),
        in_specs=[
            pl.BlockSpec((BE, 16), lambda e: (e, 0)),
            pl.BlockSpec((BE, 16), lambda e: (e, 0)),
            pl.BlockSpec((BE, 3), lambda e: (e, 0)),
            pl.BlockSpec((64, NUM_BESSEL), lambda e: (0, 0)),
            pl.BlockSpec((64, 64), lambda e: (0, 0)),
            pl.BlockSpec((CL, 64), lambda e: (0, 0)),
            pl.BlockSpec((CL, 16), lambda e: (0, 0)),
            pl.BlockSpec((CL, C), lambda e: (0, 0)),
            pl.BlockSpec((C, NUM_ELEMENTS), lambda e: (0, 0)),
        ],
        out_specs=[
            pl.BlockSpec((2, BE, HALF), lambda e: (0, e, 0)),
            pl.BlockSpec((BE, 16), lambda e: (e, 0)),
            pl.BlockSpec((BE, NUM_BESSEL), lambda e: (e, 0)),
        ],
        out_shape=[
            jax.ShapeDtypeStruct((2, E, HALF), jnp.float32),
            jax.ShapeDtypeStruct((E, 16), jnp.float32),
            jax.ShapeDtypeStruct((E, NUM_BESSEL), jnp.float32),
        ],
    )(gs, gr, shifts, w1t, w2t, w3t, s_t, t_t, wupt)


def _edge1_body(g1_ref, y_ref, ef_ref, w1_ref, w2_ref, w3_ref, s_ref, t_ref,
                m_ref):
    src = g1_ref[...].astype(jnp.float32)
    Y = y_ref[...]
    R = _radial(ef_ref[...], w1_ref[...], w2_ref[...], w3_ref[...])
    m = R * ((src[:, 0:C] @ t_ref[...]) * (Y @ s_ref[...]) + src)
    m_ref[0] = m[:, :HALF]
    m_ref[1] = m[:, HALF:]


def _edge1(g1, y_sto, ef_sto, w1, w2, w3p, s_c, t_c):
    return pl.pallas_call(
        _edge1_body,
        grid=(GE,),
        in_specs=[
            pl.BlockSpec((BE, CL), lambda e: (e, 0)),
            pl.BlockSpec((BE, 16), lambda e: (e, 0)),
            pl.BlockSpec((BE, NUM_BESSEL), lambda e: (e, 0)),
            pl.BlockSpec((NUM_BESSEL, 64), lambda e: (0, 0)),
            pl.BlockSpec((64, 64), lambda e: (0, 0)),
            pl.BlockSpec((64, CL), lambda e: (0, 0)),
            pl.BlockSpec((16, CL), lambda e: (0, 0)),
            pl.BlockSpec((C, CL), lambda e: (0, 0)),
        ],
        out_specs=pl.BlockSpec((2, BE, HALF), lambda e: (0, e, 0)),
        out_shape=jax.ShapeDtypeStruct((2, E, HALF), jnp.float32),
    )(g1, y_sto, ef_sto, w1, w2, w3p, s_c, t_c)


def _poly_block(A, attrs, pw1, pw2, pw3, t_c):
    s = A[:, 0:C]
    w1 = attrs @ pw1
    w2 = attrs @ pw2
    w3 = attrs @ pw3
    g = 1.0 + w2 * s + w3 * s * s
    B = A * (g @ t_c)
    add0 = jnp.concatenate([w1 * s, jnp.zeros((A.shape[0], CL - C), jnp.float32)],
                           axis=1)
    return B + add0


def _node0_body(a_ref, attrs_ref, oht_ref, wmix_ref, pw1_ref, pw2_ref, pw3_ref,
                wread_ref, ae_ref, linup_ref, t_ref,
                feats_ref, hup_ref, nout_ref, en_ref):
    step = pl.program_id(0)
    A = jnp.concatenate([a_ref[0], a_ref[1]], axis=1) @ wmix_ref[...]
    attrs = attrs_ref[...]
    feats = _poly_block(A, attrs, pw1_ref[...], pw2_ref[...], pw3_ref[...],
                        t_ref[...])
    feats_ref[...] = feats
    hup_ref[...] = (feats @ linup_ref[...]).astype(jnp.bfloat16)
    nout = feats[:, 0:C] @ wread_ref[...]
    nout_ref[...] = nout
    ne0 = attrs @ ae_ref[...]
    en_mat = nout[:, 0:N_ENERGIES] + ne0

    @pl.when(step == 0)
    def _():
        en_ref[...] = jnp.zeros_like(en_ref)

    en_ref[...] += oht_ref[0] @ en_mat


def _node0(araw, node_attrs, oht, wmixbd, pw1, pw2, pw3, wread0, ae, linupbd,
           t_c):
    return pl.pallas_call(
        _node0_body,
        grid=(GN,),
        in_specs=[
            pl.BlockSpec((2, BN, HALF), lambda n: (0, n, 0)),
            pl.BlockSpec((BN, NUM_ELEMENTS), lambda n: (n, 0)),
            pl.BlockSpec((1, NUM_GRAPHS, BN), lambda n: (n, 0, 0)),
            pl.BlockSpec((CL, CL), lambda n: (0, 0)),
            pl.BlockSpec((NUM_ELEMENTS, C), lambda n: (0, 0)),
            pl.BlockSpec((NUM_ELEMENTS, C), lambda n: (0, 0)),
            pl.BlockSpec((NUM_ELEMENTS, C), lambda n: (0, 0)),
            pl.BlockSpec((C, READ_DIM), lambda n: (0, 0)),
            pl.BlockSpec((NUM_ELEMENTS, 1), lambda n: (0, 0)),
            pl.BlockSpec((CL, CL), lambda n: (0, 0)),
            pl.BlockSpec((C, CL), lambda n: (0, 0)),
        ],
        out_specs=[
            pl.BlockSpec((BN, CL), lambda n: (n, 0)),
            pl.BlockSpec((BN, CL), lambda n: (n, 0)),
            pl.BlockSpec((BN, READ_DIM), lambda n: (n, 0)),
            pl.BlockSpec((NUM_GRAPHS, N_ENERGIES), lambda n: (0, 0)),
        ],
        out_shape=[
            jax.ShapeDtypeStruct((N, CL), jnp.float32),
            jax.ShapeDtypeStruct((N, CL), jnp.bfloat16),
            jax.ShapeDtypeStruct((N, READ_DIM), jnp.float32),
            jax.ShapeDtypeStruct((NUM_GRAPHS, N_ENERGIES), jnp.float32),
        ],
    )(araw, node_attrs, oht, wmixbd, pw1, pw2, pw3, wread0, ae, linupbd, t_c)


def _node1_body(a_ref, attrs_ref, oht_ref, wmix_ref, pw1_ref, pw2_ref, pw3_ref,
                wsc_ref, f0_ref, n0_ref, wra_ref, wrb_ref, t_ref, en0_ref,
                en_ref, nacs_ref):
    step = pl.program_id(0)
    A = jnp.concatenate([a_ref[0], a_ref[1]], axis=1) @ wmix_ref[...]
    attrs = attrs_ref[...]
    B = _poly_block(A, attrs, pw1_ref[...], pw2_ref[...], pw3_ref[...],
                    t_ref[...])
    wscn = attrs @ wsc_ref[...]
    feats = B + f0_ref[...] * (wscn @ t_ref[...])
    nout = _silu(feats[:, 0:C] @ wra_ref[...]) @ wrb_ref[...]
    nacs_ref[...] = nout[:, N_ENERGIES:READ_DIM] + n0_ref[...][:, N_ENERGIES:READ_DIM]

    @pl.when(step == 0)
    def _():
        en_ref[...] = en0_ref[...]

    en_ref[...] += oht_ref[0] @ nout[:, 0:N_ENERGIES]


def _node1(araw, node_attrs, oht, wmixbd, pw1, pw2, pw3, wsc, feats0, nout0,
           wread1a, wread1b, t_c, en0):
    return pl.pallas_call(
        _node1_body,
        grid=(GN,),
        in_specs=[
            pl.BlockSpec((2, BN, HALF), lambda n: (0, n, 0)),
            pl.BlockSpec((BN, NUM_ELEMENTS), lambda n: (n, 0)),
            pl.BlockSpec((1, NUM_GRAPHS, BN), lambda n: (n, 0, 0)),
            pl.BlockSpec((CL, CL), lambda n: (0, 0)),
            pl.BlockSpec((NUM_ELEMENTS, C), lambda n: (0, 0)),
            pl.BlockSpec((NUM_ELEMENTS, C), lambda n: (0, 0)),
            pl.BlockSpec((NUM_ELEMENTS, C), lambda n: (0, 0)),
            pl.BlockSpec((NUM_ELEMENTS, C), lambda n: (0, 0)),
            pl.BlockSpec((BN, CL), lambda n: (n, 0)),
            pl.BlockSpec((BN, READ_DIM), lambda n: (n, 0)),
            pl.BlockSpec((C, 16), lambda n: (0, 0)),
            pl.BlockSpec((16, READ_DIM), lambda n: (0, 0)),
            pl.BlockSpec((C, CL), lambda n: (0, 0)),
            pl.BlockSpec((NUM_GRAPHS, N_ENERGIES), lambda n: (0, 0)),
        ],
        out_specs=[
            pl.BlockSpec((NUM_GRAPHS, N_ENERGIES), lambda n: (0, 0)),
            pl.BlockSpec((BN, READ_DIM - N_ENERGIES), lambda n: (n, 0)),
        ],
        out_shape=[
            jax.ShapeDtypeStruct((NUM_GRAPHS, N_ENERGIES), jnp.float32),
            jax.ShapeDtypeStruct((N, READ_DIM - N_ENERGIES), jnp.float32),
        ],
    )(araw, node_attrs, oht, wmixbd, pw1, pw2, pw3, wsc, feats0, nout0,
      wread1a, wread1b, t_c, en0)


NW = 32
CH = 128
BLK = 2 * CH
NBLK = E // BLK
ROWS_T = N // 16


def _paired_loop(per, body2):
    assert per % 2 == 0

    def it(q, carry):
        body2(q, 0)
        body2(q, 1)
        return carry

    lax.fori_loop(0, per // 2, it, 0)


@functools.cache
def _sc_kernels():
    mesh = plsc.VectorSubcoreMesh(core_axis_name="c", subcore_axis_name="s")

    @functools.partial(
        pl.kernel,
        out_type=(
            jax.ShapeDtypeStruct((E, 16), jnp.float32),
            jax.ShapeDtypeStruct((E, 16), jnp.float32),
        ),
        mesh=mesh,
        compiler_params=pltpu.CompilerParams(use_tc_tiling_on_sc=False),
        scratch_types=[
            pltpu.VMEM((2, 2, CH), jnp.int32),
            pltpu.VMEM((2, 2, CH), jnp.int32),
            pltpu.VMEM((2, BLK, 16), jnp.float32),
            pltpu.VMEM((2, BLK, 16), jnp.float32),
            [pltpu.SemaphoreType.DMA] * 2,
            [pltpu.SemaphoreType.DMA] * 2,
            [pltpu.SemaphoreType.DMA] * 2,
        ],
    )
    def gather0(t0_hbm, snd_hbm, rcv_hbm, gs_hbm, gr_hbm,
                idx_s, idx_r, buf_s, buf_r, semi, semg, semw):
        wid = lax.axis_index("s") * 2 + lax.axis_index("c")
        per = 20

        def blk(i):
            return wid + NW * i

        def issue_idx(i, p):
            @pl.when(blk(i) < NBLK)
            def _():
                pltpu.async_copy(snd_hbm.at[pl.ds(2 * blk(i), 2)],
                                 idx_s.at[p], semi[p])
                pltpu.async_copy(rcv_hbm.at[pl.ds(2 * blk(i), 2)],
                                 idx_r.at[p], semi[p])

        issue_idx(0, 0)
        issue_idx(1, 1)

        def body2(q, p):
            i = 2 * q + p
            b = blk(i)

            @pl.when(b < NBLK)
            def _():
                pltpu.make_async_copy(snd_hbm.at[pl.ds(0, 2)],
                                      idx_s.at[p], semi[p]).wait()
                pltpu.make_async_copy(snd_hbm.at[pl.ds(0, 2)],
                                      idx_r.at[p], semi[p]).wait()

                @pl.when(q >= 1)
                def _():
                    pltpu.make_async_copy(gs_hbm.at[pl.ds(0, BLK)],
                                          buf_s.at[p], semw[p]).wait()
                    pltpu.make_async_copy(gr_hbm.at[pl.ds(0, BLK)],
                                          buf_r.at[p], semw[p]).wait()
                for j in range(2):
                    pltpu.async_copy(t0_hbm.at[idx_s.at[p, j]],
                                     buf_s.at[p, pl.ds(j * CH, CH)], semg[p])
                    pltpu.async_copy(t0_hbm.at[idx_r.at[p, j]],
                                     buf_r.at[p, pl.ds(j * CH, CH)], semg[p])
                for j in range(2):
                    pltpu.make_async_copy(t0_hbm.at[idx_s.at[p, j]],
                                          buf_s.at[p, pl.ds(j * CH, CH)],
                                          semg[p]).wait()
                    pltpu.make_async_copy(t0_hbm.at[idx_r.at[p, j]],
                                          buf_r.at[p, pl.ds(j * CH, CH)],
                                          semg[p]).wait()
                pltpu.async_copy(buf_s.at[p],
                                 gs_hbm.at[pl.ds(b * BLK, BLK)], semw[p])
                pltpu.async_copy(buf_r.at[p],
                                 gr_hbm.at[pl.ds(b * BLK, BLK)], semw[p])
                issue_idx(i + 2, p)

        _paired_loop(per, body2)

        for p in range(2):
            @pl.when(blk(p) < NBLK)
            def _():
                pltpu.make_async_copy(gs_hbm.at[pl.ds(0, BLK)],
                                      buf_s.at[p], semw[p]).wait()
                pltpu.make_async_copy(gr_hbm.at[pl.ds(0, BLK)],
                                      buf_r.at[p], semw[p]).wait()

    @functools.partial(
        pl.kernel,
        out_type=jax.ShapeDtypeStruct((E, CL), jnp.bfloat16),
        mesh=mesh,
        compiler_params=pltpu.CompilerParams(use_tc_tiling_on_sc=False),
        scratch_types=[
            pltpu.VMEM((2, 1, CH), jnp.int32),
            pltpu.VMEM((2, CH, CL), jnp.bfloat16),
            [pltpu.SemaphoreType.DMA] * 2,
            [pltpu.SemaphoreType.DMA] * 2,
            [pltpu.SemaphoreType.DMA] * 2,
        ],
    )
    def gather1(tab_hbm, snd_hbm, out_hbm, idx_s, buf, semi, semg, semw):
        wid = lax.axis_index("s") * 2 + lax.axis_index("c")
        per = 40
        nch = E // CH

        def issue_idx(i, p):
            @pl.when(wid + NW * i < nch)
            def _():
                pltpu.async_copy(snd_hbm.at[pl.ds(wid + NW * i, 1)],
                                 idx_s.at[p], semi[p])

        issue_idx(0, 0)
        issue_idx(1, 1)

        def body2(q, p):
            i = 2 * q + p
            k = wid + NW * i

            @pl.when(k < nch)
            def _():
                pltpu.make_async_copy(snd_hbm.at[pl.ds(0, 1)],
                                      idx_s.at[p], semi[p]).wait()

                @pl.when(q >= 1)
                def _():
                    pltpu.make_async_copy(out_hbm.at[pl.ds(0, CH)],
                                          buf.at[p], semw[p]).wait()
                pltpu.async_copy(tab_hbm.at[idx_s.at[p, 0]], buf.at[p],
                                 semg[p])
                pltpu.make_async_copy(tab_hbm.at[idx_s.at[p, 0]], buf.at[p],
                                      semg[p]).wait()
                pltpu.async_copy(buf.at[p], out_hbm.at[pl.ds(k * CH, CH)],
                                 semw[p])
                issue_idx(i + 2, p)

        _paired_loop(per, body2)

        for p in range(2):
            @pl.when(wid + NW * p < nch)
            def _():
                pltpu.make_async_copy(out_hbm.at[pl.ds(0, CH)],
                                      buf.at[p], semw[p]).wait()

    @functools.partial(
        pl.kernel,
        out_type=jax.ShapeDtypeStruct((2, N, HALF), jnp.float32),
        mesh=mesh,
        compiler_params=pltpu.CompilerParams(use_tc_tiling_on_sc=False),
        scratch_types=[
            pltpu.VMEM((2, 1, CH), jnp.int32),
            pltpu.VMEM((2, CH, HALF), jnp.float32),
            pltpu.VMEM_SHARED((N, HALF), jnp.float32),
            [pltpu.SemaphoreType.DMA] * 2,
            [pltpu.SemaphoreType.DMA] * 2,
        ],
    )
    def scatter(m_hbm, rcv_hbm, zeros_hbm, a_hbm, idx_v, row_buf, acc,
                seml, sems):
        cid = lax.axis_index("c")
        sid = lax.axis_index("s")
        per = 80
        nch = E // CH

        def issue_loads(i, p):
            k = sid + 16 * i

            @pl.when(k < nch)
            def _():
                pltpu.async_copy(rcv_hbm.at[pl.ds(k, 1)],
                                 idx_v.at[p], seml[p])
                pltpu.async_copy(m_hbm.at[cid, pl.ds(k * CH, CH)],
                                 row_buf.at[p], seml[p])

        pltpu.sync_copy(zeros_hbm, acc.at[pl.ds(sid * ROWS_T, ROWS_T)])
        plsc.subcore_barrier()
        issue_loads(0, 0)
        issue_loads(1, 1)

        def body2(q, p):
            i = 2 * q + p
            k = sid + 16 * i

            @pl.when(k < nch)
            def _():
                pltpu.make_async_copy(rcv_hbm.at[pl.ds(0, 1)],
                                      idx_v.at[p], seml[p]).wait()
                pltpu.make_async_copy(m_hbm.at[0, pl.ds(0, CH)],
                                      row_buf.at[p], seml[p]).wait()
                pltpu.async_copy(row_buf.at[p], acc.at[idx_v.at[p, 0]],
                                 sems[p], add=True).wait()
                issue_loads(i + 2, p)

        _paired_loop(per, body2)
        plsc.subcore_barrier()
        pltpu.sync_copy(acc.at[pl.ds(sid * ROWS_T, ROWS_T)],
                        a_hbm.at[cid, pl.ds(sid * ROWS_T, ROWS_T)])

    return gather0, gather1, scatter


def _gather0(tbl, snd, rcv):
    return _sc_kernels()[0](tbl, snd, rcv)


def _gather1(tab, snd):
    return _sc_kernels()[1](tab, snd)


def _scatter(m2, rcv, zeros_t):
    return _sc_kernels()[2](m2, rcv, zeros_t)


def kernel(positions, node_attrs, shifts, params, edge_index, batch, ptr):
    p = params
    sender = edge_index[0].reshape(E // CH, CH)
    receiver = edge_index[1].reshape(E // CH, CH)

    eye9 = jnp.eye(L, dtype=jnp.float32)
    wmixbd0 = jnp.kron(eye9, p['W_mix_0']) * (1.0 / AVG_NEIGH)
    wmixbd1 = jnp.kron(eye9, p['W_mix_1']) * (1.0 / AVG_NEIGH)
    linup1bd = jnp.kron(eye9, p['lin_up_1'])
    w3p0 = p['rW3_0'][:, _W3PERM]
    w3p1 = p['rW3_1'][:, _W3PERM]
    wemb_up0 = p['W_embed'] @ p['lin_up_0']
    iota_col = jnp.arange(NUM_ELEMENTS, dtype=jnp.float32).reshape(NUM_ELEMENTS, 1)
    s_c = jnp.asarray(_S)
    t_c = jnp.asarray(_T)
    oht = jnp.transpose(jax.nn.one_hot(batch, NUM_GRAPHS, dtype=jnp.float32))
    oht = oht.reshape(NUM_GRAPHS, GN, BN).transpose(1, 0, 2)
    ae = p['atomic_energies'].reshape(NUM_ELEMENTS, 1)
    zeros_t = jnp.zeros((ROWS_T, HALF), jnp.float32)

    tbl = _prep(positions, node_attrs, iota_col)
    gs, gr = _gather0(tbl, sender, receiver)
    m2, y_sto, ef_sto = _edge0(gs, gr, shifts, p['rW1_0'].T, p['rW2_0'].T,
                               w3p0.T, s_c.T, t_c.T, wemb_up0.T)
    araw0 = _scatter(m2, receiver, zeros_t)
    feats0, hup1, nout0, en0 = _node0(araw0, node_attrs, oht, wmixbd0,
                                      p['pw1_0'], p['pw2_0'], p['pw3_0'],
                                      p['W_read0'], ae, linup1bd, t_c)
    g1 = _gather1(hup1, sender)
    m2b = _edge1(g1, y_sto, ef_sto, p['rW1_1'], p['rW2_1'], w3p1, s_c, t_c)
    araw1 = _scatter(m2b, receiver, zeros_t)
    en, nacs9 = _node1(araw1, node_attrs, oht, wmixbd1,
                       p['pw1_1'], p['pw2_1'], p['pw3_1'], p['wsc_1'],
                       feats0, nout0, p['W_read1a'], p['W_read1b'], t_c, en0)
    return en, nacs9.reshape(N, N_ENERGIES, 3)


# speedup vs baseline: 1.0754x; 1.0754x over previous
import functools

import jax
import jax.numpy as jnp
import numpy as np
from jax import lax
from jax.experimental import pallas as pl
from jax.experimental.pallas import tpu as pltpu
from jax.experimental.pallas import tpu_sc as plsc

N = 10000
E = 160000
NUM_ELEMENTS = 4
C = 32
L = 9
NUM_BESSEL = 8
R_MAX = 5.0
NUM_GRAPHS = 16
N_ENERGIES = 3
AVG_NEIGH = 16.0
READ_DIM = 12
CL = C * L
HALF = CL // 2

BE = 2000
BN = 2000
GE = E // BE
GN = N // BN

_S = np.zeros((16, CL), np.float32)
_T = np.zeros((C, CL), np.float32)
for _l in range(L):
    for _c in range(C):
        _S[_l, _l * C + _c] = 1.0
        _T[_c, _l * C + _c] = 1.0
_W3PERM = np.array([c * L + l for l in range(L) for c in range(C)], np.int32)

_SQ3 = 1.7320508075688772
_SQ5 = 2.23606797749979
_SQ15 = 3.872983346207417


def _silu(x):
    return x * (1.0 / (1.0 + jnp.exp(-x)))


def _prep_body(pos_ref, attrs_ref, iv_ref, tbl_ref):
    pos = pos_ref[...]
    sp = attrs_ref[...] @ iv_ref[...]
    tbl_ref[...] = jnp.concatenate(
        [pos, sp, jnp.zeros((BN, 12), jnp.float32)], axis=1)


def _prep(positions, node_attrs, iota_col):
    return pl.pallas_call(
        _prep_body,
        grid=(GN,),
        in_specs=[
            pl.BlockSpec((BN, 3), lambda n: (n, 0)),
            pl.BlockSpec((BN, NUM_ELEMENTS), lambda n: (n, 0)),
            pl.BlockSpec((NUM_ELEMENTS, 1), lambda n: (0, 0)),
        ],
        out_specs=pl.BlockSpec((BN, 16), lambda n: (n, 0)),
        out_shape=jax.ShapeDtypeStruct((N, 16), jnp.float32),
    )(positions, node_attrs, iota_col)


def _geom_t(vx, vy, vz):
    r = jnp.sqrt(vx * vx + vy * vy + vz * vz + 1e-18)
    inv = 1.0 / r
    x, y, z = vx * inv, vy * inv, vz * inv
    ones = jnp.ones_like(x)
    YT = jnp.concatenate([
        ones, _SQ3 * x, _SQ3 * y, _SQ3 * z, _SQ15 * x * y, _SQ15 * y * z,
        0.5 * _SQ5 * (3.0 * z * z - 1.0), _SQ15 * x * z,
        0.5 * _SQ15 * (x * x - y * y),
        jnp.zeros((7, x.shape[1]), jnp.float32),
    ], axis=0)
    rr = jnp.maximum(r, 1e-9)
    xx = (jnp.pi / R_MAX) * rr
    s1 = jnp.sin(xx)
    c2 = 2.0 * jnp.cos(xx)
    sins = [s1, c2 * s1]
    for _ in range(NUM_BESSEL - 2):
        sins.append(c2 * sins[-1] - sins[-2])
    bes = jnp.sqrt(2.0 / R_MAX) * jnp.concatenate(sins, axis=0) / rr
    u = jnp.minimum(r * (1.0 / R_MAX), 1.0)
    u2 = u * u
    u5 = u2 * u2 * u
    f = 1.0 - 21.0 * u5 + 35.0 * u5 * u - 15.0 * u5 * u2
    fc = jnp.where(r < R_MAX, f, 0.0)
    return YT, bes * fc


def _radial(ef, w1, w2, w3):
    r1 = _silu(ef @ w1)
    r2 = _silu(r1 @ w2)
    return r2 @ w3


def _edge0_body(gs_ref, gr_ref, sh_ref, w1t_ref, w2t_ref, w3t_ref, st_ref,
                tt_ref, wupt_ref, m_ref, y_ref, ef_ref):
    gst = jnp.transpose(gs_ref[...][:, 0:4])
    grt = jnp.transpose(gr_ref[...][:, 0:3])
    sht = jnp.transpose(sh_ref[...])
    dps = grt - gst[0:3] + sht
    YT, efT = _geom_t(dps[0:1], dps[1:2], dps[2:3])
    r1 = _silu(w1t_ref[...] @ efT)
    r2 = _silu(w2t_ref[...] @ r1)
    RT = w3t_ref[...] @ r2
    spi = gst[3:4].astype(jnp.int32)
    oh = jnp.where(
        lax.broadcasted_iota(jnp.int32, (NUM_ELEMENTS, spi.shape[1]), 0)
        == spi, 1.0, 0.0)
    hT = wupt_ref[...] @ oh
    mT = RT * (tt_ref[...] @ hT) * (st_ref[...] @ YT)
    m = jnp.transpose(mT)
    m_ref[0] = m[:, :HALF]
    m_ref[1] = m[:, HALF:]
    y_ref[...] = jnp.transpose(YT)
    ef_ref[...] = jnp.transpose(efT)


def _edge0(gs, gr, shifts, w1t, w2t, w3t, s_t, t_t, wupt):
    return pl.pallas_call(
        _edge0_body,
        grid=(GE,),
        in_specs=[
            pl.BlockSpec((BE, 16), lambda e: (e, 0)),
            pl.BlockSpec((BE, 16), lambda e: (e, 0)),
            pl.BlockSpec((BE, 3), lambda e: (e, 0)),
            pl.BlockSpec((64, NUM_BESSEL), lambda e: (0, 0)),
            pl.BlockSpec((64, 64), lambda e: (0, 0)),
            pl.BlockSpec((CL, 64), lambda e: (0, 0)),
            pl.BlockSpec((CL, 16), lambda e: (0, 0)),
            pl.BlockSpec((CL, C), lambda e: (0, 0)),
            pl.BlockSpec((C, NUM_ELEMENTS), lambda e: (0, 0)),
        ],
        out_specs=[
            pl.BlockSpec((2, BE, HALF), lambda e: (0, e, 0)),
            pl.BlockSpec((BE, 16), lambda e: (e, 0)),
            pl.BlockSpec((BE, NUM_BESSEL), lambda e: (e, 0)),
        ],
        out_shape=[
            jax.ShapeDtypeStruct((2, E, HALF), jnp.float32),
            jax.ShapeDtypeStruct((E, 16), jnp.float32),
            jax.ShapeDtypeStruct((E, NUM_BESSEL), jnp.float32),
        ],
    )(gs, gr, shifts, w1t, w2t, w3t, s_t, t_t, wupt)


def _edge1_body(g1_ref, y_ref, ef_ref, w1_ref, w2_ref, w3_ref, s_ref, t_ref,
                m_ref):
    src = g1_ref[...]
    Y = y_ref[...]
    R = _radial(ef_ref[...], w1_ref[...], w2_ref[...], w3_ref[...])
    m = R * ((src[:, 0:C] @ t_ref[...]) * (Y @ s_ref[...]) + src)
    m_ref[0] = m[:, :HALF]
    m_ref[1] = m[:, HALF:]


def _edge1(g1, y_sto, ef_sto, w1, w2, w3p, s_c, t_c):
    return pl.pallas_call(
        _edge1_body,
        grid=(GE,),
        in_specs=[
            pl.BlockSpec((BE, CL), lambda e: (e, 0)),
            pl.BlockSpec((BE, 16), lambda e: (e, 0)),
            pl.BlockSpec((BE, NUM_BESSEL), lambda e: (e, 0)),
            pl.BlockSpec((NUM_BESSEL, 64), lambda e: (0, 0)),
            pl.BlockSpec((64, 64), lambda e: (0, 0)),
            pl.BlockSpec((64, CL), lambda e: (0, 0)),
            pl.BlockSpec((16, CL), lambda e: (0, 0)),
            pl.BlockSpec((C, CL), lambda e: (0, 0)),
        ],
        out_specs=pl.BlockSpec((2, BE, HALF), lambda e: (0, e, 0)),
        out_shape=jax.ShapeDtypeStruct((2, E, HALF), jnp.float32),
    )(g1, y_sto, ef_sto, w1, w2, w3p, s_c, t_c)


def _poly_block(A, attrs, pw1, pw2, pw3, t_c):
    s = A[:, 0:C]
    w1 = attrs @ pw1
    w2 = attrs @ pw2
    w3 = attrs @ pw3
    g = 1.0 + w2 * s + w3 * s * s
    B = A * (g @ t_c)
    add0 = jnp.concatenate([w1 * s, jnp.zeros((A.shape[0], CL - C), jnp.float32)],
                           axis=1)
    return B + add0


def _node0_body(a_ref, attrs_ref, oht_ref, wmix_ref, pw1_ref, pw2_ref, pw3_ref,
                wread_ref, ae_ref, linup_ref, t_ref,
                feats_ref, hup_ref, nout_ref, en_ref):
    step = pl.program_id(0)
    A = jnp.concatenate([a_ref[0], a_ref[1]], axis=1) @ wmix_ref[...]
    attrs = attrs_ref[...]
    feats = _poly_block(A, attrs, pw1_ref[...], pw2_ref[...], pw3_ref[...],
                        t_ref[...])
    feats_ref[...] = feats
    hup_ref[...] = feats @ linup_ref[...]
    nout = feats[:, 0:C] @ wread_ref[...]
    nout_ref[...] = nout
    ne0 = attrs @ ae_ref[...]
    en_mat = nout[:, 0:N_ENERGIES] + ne0

    @pl.when(step == 0)
    def _():
        en_ref[...] = jnp.zeros_like(en_ref)

    en_ref[...] += oht_ref[0] @ en_mat


def _node0(araw, node_attrs, oht, wmixbd, pw1, pw2, pw3, wread0, ae, linupbd,
           t_c):
    return pl.pallas_call(
        _node0_body,
        grid=(GN,),
        in_specs=[
            pl.BlockSpec((2, BN, HALF), lambda n: (0, n, 0)),
            pl.BlockSpec((BN, NUM_ELEMENTS), lambda n: (n, 0)),
            pl.BlockSpec((1, NUM_GRAPHS, BN), lambda n: (n, 0, 0)),
            pl.BlockSpec((CL, CL), lambda n: (0, 0)),
            pl.BlockSpec((NUM_ELEMENTS, C), lambda n: (0, 0)),
            pl.BlockSpec((NUM_ELEMENTS, C), lambda n: (0, 0)),
            pl.BlockSpec((NUM_ELEMENTS, C), lambda n: (0, 0)),
            pl.BlockSpec((C, READ_DIM), lambda n: (0, 0)),
            pl.BlockSpec((NUM_ELEMENTS, 1), lambda n: (0, 0)),
            pl.BlockSpec((CL, CL), lambda n: (0, 0)),
            pl.BlockSpec((C, CL), lambda n: (0, 0)),
        ],
        out_specs=[
            pl.BlockSpec((BN, CL), lambda n: (n, 0)),
            pl.BlockSpec((BN, CL), lambda n: (n, 0)),
            pl.BlockSpec((BN, READ_DIM), lambda n: (n, 0)),
            pl.BlockSpec((NUM_GRAPHS, N_ENERGIES), lambda n: (0, 0)),
        ],
        out_shape=[
            jax.ShapeDtypeStruct((N, CL), jnp.float32),
            jax.ShapeDtypeStruct((N, CL), jnp.float32),
            jax.ShapeDtypeStruct((N, READ_DIM), jnp.float32),
            jax.ShapeDtypeStruct((NUM_GRAPHS, N_ENERGIES), jnp.float32),
        ],
    )(araw, node_attrs, oht, wmixbd, pw1, pw2, pw3, wread0, ae, linupbd, t_c)


def _node1_body(a_ref, attrs_ref, oht_ref, wmix_ref, pw1_ref, pw2_ref, pw3_ref,
                wsc_ref, f0_ref, n0_ref, wra_ref, wrb_ref, t_ref, en0_ref,
                en_ref, nacs_ref):
    step = pl.program_id(0)
    A = jnp.concatenate([a_ref[0], a_ref[1]], axis=1) @ wmix_ref[...]
    attrs = attrs_ref[...]
    B = _poly_block(A, attrs, pw1_ref[...], pw2_ref[...], pw3_ref[...],
                    t_ref[...])
    wscn = attrs @ wsc_ref[...]
    feats = B + f0_ref[...] * (wscn @ t_ref[...])
    nout = _silu(feats[:, 0:C] @ wra_ref[...]) @ wrb_ref[...]
    nacs_ref[...] = nout[:, N_ENERGIES:READ_DIM] + n0_ref[...][:, N_ENERGIES:READ_DIM]

    @pl.when(step == 0)
    def _():
        en_ref[...] = en0_ref[...]

    en_ref[...] += oht_ref[0] @ nout[:, 0:N_ENERGIES]


def _node1(araw, node_attrs, oht, wmixbd, pw1, pw2, pw3, wsc, feats0, nout0,
           wread1a, wread1b, t_c, en0):
    return pl.pallas_call(
        _node1_body,
        grid=(GN,),
        in_specs=[
            pl.BlockSpec((2, BN, HALF), lambda n: (0, n, 0)),
            pl.BlockSpec((BN, NUM_ELEMENTS), lambda n: (n, 0)),
            pl.BlockSpec((1, NUM_GRAPHS, BN), lambda n: (n, 0, 0)),
            pl.BlockSpec((CL, CL), lambda n: (0, 0)),
            pl.BlockSpec((NUM_ELEMENTS, C), lambda n: (0, 0)),
            pl.BlockSpec((NUM_ELEMENTS, C), lambda n: (0, 0)),
            pl.BlockSpec((NUM_ELEMENTS, C), lambda n: (0, 0)),
            pl.BlockSpec((NUM_ELEMENTS, C), lambda n: (0, 0)),
            pl.BlockSpec((BN, CL), lambda n: (n, 0)),
            pl.BlockSpec((BN, READ_DIM), lambda n: (n, 0)),
            pl.BlockSpec((C, 16), lambda n: (0, 0)),
            pl.BlockSpec((16, READ_DIM), lambda n: (0, 0)),
            pl.BlockSpec((C, CL), lambda n: (0, 0)),
            pl.BlockSpec((NUM_GRAPHS, N_ENERGIES), lambda n: (0, 0)),
        ],
        out_specs=[
            pl.BlockSpec((NUM_GRAPHS, N_ENERGIES), lambda n: (0, 0)),
            pl.BlockSpec((BN, READ_DIM - N_ENERGIES), lambda n: (n, 0)),
        ],
        out_shape=[
            jax.ShapeDtypeStruct((NUM_GRAPHS, N_ENERGIES), jnp.float32),
            jax.ShapeDtypeStruct((N, READ_DIM - N_ENERGIES), jnp.float32),
        ],
    )(araw, node_attrs, oht, wmixbd, pw1, pw2, pw3, wsc, feats0, nout0,
      wread1a, wread1b, t_c, en0)


NW = 32
CH = 128
BLK = 2 * CH
NBLK = E // BLK
ROWS_T = N // 16


def _paired_loop(per, body2):
    assert per % 2 == 0

    def it(q, carry):
        body2(q, 0)
        body2(q, 1)
        return carry

    lax.fori_loop(0, per // 2, it, 0)


@functools.cache
def _sc_kernels():
    mesh = plsc.VectorSubcoreMesh(core_axis_name="c", subcore_axis_name="s")

    @functools.partial(
        pl.kernel,
        out_type=(
            jax.ShapeDtypeStruct((E, 16), jnp.float32),
            jax.ShapeDtypeStruct((E, 16), jnp.float32),
        ),
        mesh=mesh,
        compiler_params=pltpu.CompilerParams(use_tc_tiling_on_sc=False),
        scratch_types=[
            pltpu.VMEM((2, 2, CH), jnp.int32),
            pltpu.VMEM((2, 2, CH), jnp.int32),
            pltpu.VMEM((2, BLK, 16), jnp.float32),
            pltpu.VMEM((2, BLK, 16), jnp.float32),
            [pltpu.SemaphoreType.DMA] * 2,
            [pltpu.SemaphoreType.DMA] * 2,
            [pltpu.SemaphoreType.DMA] * 2,
        ],
    )
    def gather0(t0_hbm, snd_hbm, rcv_hbm, gs_hbm, gr_hbm,
                idx_s, idx_r, buf_s, buf_r, semi, semg, semw):
        wid = lax.axis_index("s") * 2 + lax.axis_index("c")
        per = 20

        def blk(i):
            return wid + NW * i

        def issue_idx(i, p):
            @pl.when(blk(i) < NBLK)
            def _():
                pltpu.async_copy(snd_hbm.at[pl.ds(2 * blk(i), 2)],
                                 idx_s.at[p], semi[p])
                pltpu.async_copy(rcv_hbm.at[pl.ds(2 * blk(i), 2)],
                                 idx_r.at[p], semi[p])

        issue_idx(0, 0)
        issue_idx(1, 1)

        def body2(q, p):
            i = 2 * q + p
            b = blk(i)

            @pl.when(b < NBLK)
            def _():
                pltpu.make_async_copy(snd_hbm.at[pl.ds(0, 2)],
                                      idx_s.at[p], semi[p]).wait()
                pltpu.make_async_copy(snd_hbm.at[pl.ds(0, 2)],
                                      idx_r.at[p], semi[p]).wait()

                @pl.when(q >= 1)
                def _():
                    pltpu.make_async_copy(gs_hbm.at[pl.ds(0, BLK)],
                                          buf_s.at[p], semw[p]).wait()
                    pltpu.make_async_copy(gr_hbm.at[pl.ds(0, BLK)],
                                          buf_r.at[p], semw[p]).wait()
                for j in range(2):
                    pltpu.async_copy(t0_hbm.at[idx_s.at[p, j]],
                                     buf_s.at[p, pl.ds(j * CH, CH)], semg[p])
                    pltpu.async_copy(t0_hbm.at[idx_r.at[p, j]],
                                     buf_r.at[p, pl.ds(j * CH, CH)], semg[p])
                for j in range(2):
                    pltpu.make_async_copy(t0_hbm.at[idx_s.at[p, j]],
                                          buf_s.at[p, pl.ds(j * CH, CH)],
                                          semg[p]).wait()
                    pltpu.make_async_copy(t0_hbm.at[idx_r.at[p, j]],
                                          buf_r.at[p, pl.ds(j * CH, CH)],
                                          semg[p]).wait()
                pltpu.async_copy(buf_s.at[p],
                                 gs_hbm.at[pl.ds(b * BLK, BLK)], semw[p])
                pltpu.async_copy(buf_r.at[p],
                                 gr_hbm.at[pl.ds(b * BLK, BLK)], semw[p])
                issue_idx(i + 2, p)

        _paired_loop(per, body2)

        for p in range(2):
            @pl.when(blk(p) < NBLK)
            def _():
                pltpu.make_async_copy(gs_hbm.at[pl.ds(0, BLK)],
                                      buf_s.at[p], semw[p]).wait()
                pltpu.make_async_copy(gr_hbm.at[pl.ds(0, BLK)],
                                      buf_r.at[p], semw[p]).wait()

    @functools.partial(
        pl.kernel,
        out_type=jax.ShapeDtypeStruct((E, CL), jnp.float32),
        mesh=mesh,
        compiler_params=pltpu.CompilerParams(use_tc_tiling_on_sc=False),
        scratch_types=[
            pltpu.VMEM((2, 1, CH), jnp.int32),
            pltpu.VMEM((2, CH, CL), jnp.float32),
            [pltpu.SemaphoreType.DMA] * 2,
            [pltpu.SemaphoreType.DMA] * 2,
            [pltpu.SemaphoreType.DMA] * 2,
        ],
    )
    def gather1(tab_hbm, snd_hbm, out_hbm, idx_s, buf, semi, semg, semw):
        wid = lax.axis_index("s") * 2 + lax.axis_index("c")
        per = 40
        nch = E // CH

        def issue_idx(i, p):
            @pl.when(wid + NW * i < nch)
            def _():
                pltpu.async_copy(snd_hbm.at[pl.ds(wid + NW * i, 1)],
                                 idx_s.at[p], semi[p])

        issue_idx(0, 0)
        issue_idx(1, 1)

        def body2(q, p):
            i = 2 * q + p
            k = wid + NW * i

            @pl.when(k < nch)
            def _():
                pltpu.make_async_copy(snd_hbm.at[pl.ds(0, 1)],
                                      idx_s.at[p], semi[p]).wait()

                @pl.when(q >= 1)
                def _():
                    pltpu.make_async_copy(out_hbm.at[pl.ds(0, CH)],
                                          buf.at[p], semw[p]).wait()
                pltpu.async_copy(tab_hbm.at[idx_s.at[p, 0]], buf.at[p],
                                 semg[p])
                pltpu.make_async_copy(tab_hbm.at[idx_s.at[p, 0]], buf.at[p],
                                      semg[p]).wait()
                pltpu.async_copy(buf.at[p], out_hbm.at[pl.ds(k * CH, CH)],
                                 semw[p])
                issue_idx(i + 2, p)

        _paired_loop(per, body2)

        for p in range(2):
            @pl.when(wid + NW * p < nch)
            def _():
                pltpu.make_async_copy(out_hbm.at[pl.ds(0, CH)],
                                      buf.at[p], semw[p]).wait()

    @functools.partial(
        pl.kernel,
        out_type=jax.ShapeDtypeStruct((2, N, HALF), jnp.float32),
        mesh=mesh,
        compiler_params=pltpu.CompilerParams(use_tc_tiling_on_sc=False),
        scratch_types=[
            pltpu.VMEM((2, 1, CH), jnp.int32),
            pltpu.VMEM((2, CH, HALF), jnp.float32),
            pltpu.VMEM_SHARED((N, HALF), jnp.float32),
            [pltpu.SemaphoreType.DMA] * 2,
            [pltpu.SemaphoreType.DMA] * 2,
        ],
    )
    def scatter(m_hbm, rcv_hbm, zeros_hbm, a_hbm, idx_v, row_buf, acc,
                seml, sems):
        cid = lax.axis_index("c")
        sid = lax.axis_index("s")
        per = 80
        nch = E // CH

        def issue_loads(i, p):
            k = sid + 16 * i

            @pl.when(k < nch)
            def _():
                pltpu.async_copy(rcv_hbm.at[pl.ds(k, 1)],
                                 idx_v.at[p], seml[p])
                pltpu.async_copy(m_hbm.at[cid, pl.ds(k * CH, CH)],
                                 row_buf.at[p], seml[p])

        pltpu.sync_copy(zeros_hbm, acc.at[pl.ds(sid * ROWS_T, ROWS_T)])
        plsc.subcore_barrier()
        issue_loads(0, 0)
        issue_loads(1, 1)

        def body2(q, p):
            i = 2 * q + p
            k = sid + 16 * i

            @pl.when(k < nch)
            def _():
                pltpu.make_async_copy(rcv_hbm.at[pl.ds(0, 1)],
                                      idx_v.at[p], seml[p]).wait()
                pltpu.make_async_copy(m_hbm.at[0, pl.ds(0, CH)],
                                      row_buf.at[p], seml[p]).wait()
                pltpu.async_copy(row_buf.at[p], acc.at[idx_v.at[p, 0]],
                                 sems[p], add=True).wait()
                issue_loads(i + 2, p)

        _paired_loop(per, body2)
        plsc.subcore_barrier()
        pltpu.sync_copy(acc.at[pl.ds(sid * ROWS_T, ROWS_T)],
                        a_hbm.at[cid, pl.ds(sid * ROWS_T, ROWS_T)])

    return gather0, gather1, scatter


def _gather0(tbl, snd, rcv):
    return _sc_kernels()[0](tbl, snd, rcv)


def _gather1(tab, snd):
    return _sc_kernels()[1](tab, snd)


def _scatter(m2, rcv, zeros_t):
    return _sc_kernels()[2](m2, rcv, zeros_t)


def kernel(positions, node_attrs, shifts, params, edge_index, batch, ptr):
    p = params
    sender = edge_index[0].reshape(E // CH, CH)
    receiver = edge_index[1].reshape(E // CH, CH)

    eye9 = jnp.eye(L, dtype=jnp.float32)
    wmixbd0 = jnp.kron(eye9, p['W_mix_0']) * (1.0 / AVG_NEIGH)
    wmixbd1 = jnp.kron(eye9, p['W_mix_1']) * (1.0 / AVG_NEIGH)
    linup1bd = jnp.kron(eye9, p['lin_up_1'])
    w3p0 = p['rW3_0'][:, _W3PERM]
    w3p1 = p['rW3_1'][:, _W3PERM]
    wemb_up0 = p['W_embed'] @ p['lin_up_0']
    iota_col = jnp.arange(NUM_ELEMENTS, dtype=jnp.float32).reshape(NUM_ELEMENTS, 1)
    s_c = jnp.asarray(_S)
    t_c = jnp.asarray(_T)
    oht = jnp.transpose(jax.nn.one_hot(batch, NUM_GRAPHS, dtype=jnp.float32))
    oht = oht.reshape(NUM_GRAPHS, GN, BN).transpose(1, 0, 2)
    ae = p['atomic_energies'].reshape(NUM_ELEMENTS, 1)
    zeros_t = jnp.zeros((ROWS_T, HALF), jnp.float32)

    tbl = _prep(positions, node_attrs, iota_col)
    gs, gr = _gather0(tbl, sender, receiver)
    m2, y_sto, ef_sto = _edge0(gs, gr, shifts, p['rW1_0'].T, p['rW2_0'].T,
                               w3p0.T, s_c.T, t_c.T, wemb_up0.T)
    araw0 = _scatter(m2, receiver, zeros_t)
    feats0, hup1, nout0, en0 = _node0(araw0, node_attrs, oht, wmixbd0,
                                      p['pw1_0'], p['pw2_0'], p['pw3_0'],
                                      p['W_read0'], ae, linup1bd, t_c)
    g1 = _gather1(hup1, sender)
    m2b = _edge1(g1, y_sto, ef_sto, p['rW1_1'], p['rW2_1'], w3p1, s_c, t_c)
    araw1 = _scatter(m2b, receiver, zeros_t)
    en, nacs9 = _node1(araw1, node_attrs, oht, wmixbd1,
                       p['pw1_1'], p['pw2_1'], p['pw3_1'], p['wsc_1'],
                       feats0, nout0, p['W_read1a'], p['W_read1b'], t_c, en0)
    return en, nacs9.reshape(N, N_ENERGIES, 3)
